# baseline (speedup 1.0000x reference)
import functools

import jax
import jax.numpy as jnp
import numpy as np
from jax import lax
from jax.experimental import pallas as pl
from jax.experimental.pallas import tpu as pltpu
from jax.experimental.pallas import tpu_sc as plsc

PATCH = 16
STDDEV = 2.0
KC = 16


_U32 = np.uint32


def _threefry2x32_np(k1, k2, x0, x1):
    def rotl(x, d):
        return (x << _U32(d)) | (x >> _U32(32 - d))

    rots = [(13, 15, 26, 6), (17, 29, 16, 24)]
    ks = [_U32(k1), _U32(k2), _U32(k1) ^ _U32(k2) ^ _U32(0x1BD11BDA)]
    x = [x0 + ks[0], x1 + ks[1]]
    with np.errstate(over="ignore"):
        for i in range(5):
            for r in rots[i % 2]:
                x[0] = x[0] + x[1]
                x[1] = rotl(x[1], r)
                x[1] = x[0] ^ x[1]
            x[0] = x[0] + ks[(i + 1) % 3]
            x[1] = x[1] + ks[(i + 2) % 3] + _U32(i + 1)
    return x


def _random_bits_np(k1, k2, n):
    if jax.config.jax_threefry_partitionable:
        idx = np.arange(n, dtype=np.uint64)
        c1 = (idx >> np.uint64(32)).astype(_U32)
        c2 = (idx & np.uint64(0xFFFFFFFF)).astype(_U32)
        b1, b2 = _threefry2x32_np(k1, k2, c1, c2)
        return b1 ^ b2
    odd = n % 2
    cnt = np.arange(n + odd, dtype=_U32)
    half = len(cnt) // 2
    o1, o2 = _threefry2x32_np(k1, k2, cnt[:half], cnt[half:])
    out = np.concatenate([o1, o2])
    return out[:n] if odd else out


def _erfinv_np(x):
    x = np.asarray(x, np.float64)
    w = -np.log1p(-x * x)
    ws = w - 2.5
    p_small = np.float64(2.81022636e-08)
    for c in (3.43273939e-07, -3.5233877e-06, -4.39150654e-06, 0.00021858087,
              -0.00125372503, -0.00417768164, 0.246640727, 1.50140941):
        p_small = p_small * ws + c
    wl = np.sqrt(np.maximum(w, 1e-30)) - 3.0
    p_large = np.float64(-0.000200214257)
    for c in (0.000100950558, 0.00134934322, -0.00367342844, 0.00573950773,
              -0.0076224613, 0.00943887047, 1.00167406, 2.83297682):
        p_large = p_large * wl + c
    return np.where(w < 5.0, p_small, p_large) * x


def _shift_map_np(B, H, W):
    HP, WP = H // PATCH, W // PATCH
    n = B * HP * WP
    bits = _random_bits_np(0, 42, n)
    fb = (bits >> _U32(9)) | _U32(0x3F800000)
    f = fb.view(np.float32) - np.float32(1.0)
    lo = np.float32(np.nextafter(np.float32(-1.0), np.float32(0.0)))
    hi = np.float32(1.0)
    u = np.maximum(lo, (f * (hi - lo) + lo).astype(np.float32))
    normal = np.sqrt(2.0) * _erfinv_np(u.astype(np.float64))
    m = np.abs(normal * STDDEV)
    return m.astype(np.int32).reshape(B, HP, WP)


@functools.lru_cache(maxsize=None)
def _row_shifts(B, H, W, NW, NQ, UP):
    s_np = _shift_map_np(B, H, W)
    sv = np.repeat(s_np, PATCH, axis=1)
    RPT = H // NW
    out = np.zeros((NW, B, RPT, NQ, 16), np.int32)
    for t in range(NW):
        blk = sv[:, t * RPT:(t + 1) * RPT, :]
        out[t, :, :, :, :UP] = blk.reshape(B, RPT, NQ, UP)
    return out.reshape(-1)


def kernel(inputs):
    x = inputs
    B, H, W, C = x.shape
    info = plsc.get_sparse_core_info()
    NC, NS = info.num_cores, info.num_subcores
    NW = NC * NS
    RPT = H // NW
    PW = 128
    NQ = W // PW
    UP = PW // PATCH
    T = B * RPT * NQ
    shifts = _row_shifts(B, H, W, NW, NQ, UP)
    assert int(shifts.max()) < C

    xt = jnp.transpose(x, (0, 1, 3, 2))

    mesh = plsc.VectorSubcoreMesh(core_axis_name="c", subcore_axis_name="s")

    @functools.partial(
        pl.kernel,
        mesh=mesh,
        out_type=jax.ShapeDtypeStruct((B, H, C, W), jnp.float32),
        scratch_types=[
            pltpu.VMEM((B * RPT * NQ * 16,), jnp.int32),
            pltpu.VMEM((2, C, PW), jnp.float32),
            pltpu.VMEM((2, C, PW), jnp.float32),
            pltpu.SemaphoreType.DMA,
            pltpu.SemaphoreType.DMA,
        ],
    )
    def sc_rot(x_ref, vs_ref, out_ref, vsbuf, inb, outb, sem_in, sem_out):
        wid = lax.axis_index("s") * NC + lax.axis_index("c")
        h0 = wid * RPT
        NV = B * RPT * NQ * 16
        pltpu.sync_copy(vs_ref.at[pl.ds(wid * NV, NV)], vsbuf)

        def gather_piece(b, k, q, slot):
            pltpu.async_copy(
                x_ref.at[b, h0 + k, :, pl.ds(q * PW, PW)], inb.at[slot],
                sem_in)

        def wait_gather(slot):
            pltpu.make_async_copy(
                x_ref.at[0, 0, :, pl.ds(0, PW)], inb.at[slot], sem_in).wait()

        def wait_scatter(slot):
            pltpu.make_async_copy(
                outb.at[slot], out_ref.at[0, 0, :, pl.ds(0, PW)],
                sem_out).wait()

        def advance(b, k, q):
            qn = q + 1
            wrapq = qn >= NQ
            kn = jnp.where(wrapq, k + 1, k)
            qn = jnp.where(wrapq, 0, qn)
            wrapk = kn >= RPT
            bn = jnp.where(wrapk, b + 1, b)
            kn = jnp.where(wrapk, 0, kn)
            return bn, kn, qn

        gather_piece(0, 0, 0, 0)

        def piece(t, carry):
            b, k, q = carry
            nxt = advance(b, k, q)
            slot = lax.rem(t, 2)
            nslot = lax.rem(t + 1, 2)

            @pl.when(t >= 1)
            def _():
                wait_scatter(nslot)

            @pl.when(t + 1 < T)
            def _():
                gather_piece(nxt[0], nxt[1], nxt[2], nslot)

            wait_gather(slot)
            voff = ((b * RPT + k) * NQ + q) * 16
            vv = vsbuf[pl.ds(voff, 16)]
            for u in range(UP):
                s = vv[u]
                col = u * PATCH
                for c in range(C):
                    src = c + s
                    src = jnp.where(src >= C, src - C, src)
                    outb[slot, c, pl.ds(col, PATCH)] = (
                        inb[slot, src, pl.ds(col, PATCH)])
            pltpu.async_copy(
                outb.at[slot],
                out_ref.at[b, h0 + k, :, pl.ds(q * PW, PW)], sem_out)
            return nxt

        lax.fori_loop(0, T, piece,
                      (jnp.int32(0), jnp.int32(0), jnp.int32(0)))
        wait_scatter(lax.rem(T - 1, 2))

    out = sc_rot(xt, jnp.asarray(shifts))
    return jnp.transpose(out, (0, 1, 3, 2))


# speedup vs baseline: 5.1227x; 5.1227x over previous
import functools

import jax
import jax.numpy as jnp
import numpy as np
from jax import lax
from jax.experimental import pallas as pl
from jax.experimental.pallas import tpu as pltpu
from jax.experimental.pallas import tpu_sc as plsc

PATCH = 16
STDDEV = 2.0
KC = 16


_U32 = np.uint32


def _threefry2x32_np(k1, k2, x0, x1):
    def rotl(x, d):
        return (x << _U32(d)) | (x >> _U32(32 - d))

    rots = [(13, 15, 26, 6), (17, 29, 16, 24)]
    ks = [_U32(k1), _U32(k2), _U32(k1) ^ _U32(k2) ^ _U32(0x1BD11BDA)]
    x = [x0 + ks[0], x1 + ks[1]]
    with np.errstate(over="ignore"):
        for i in range(5):
            for r in rots[i % 2]:
                x[0] = x[0] + x[1]
                x[1] = rotl(x[1], r)
                x[1] = x[0] ^ x[1]
            x[0] = x[0] + ks[(i + 1) % 3]
            x[1] = x[1] + ks[(i + 2) % 3] + _U32(i + 1)
    return x


def _random_bits_np(k1, k2, n):
    if jax.config.jax_threefry_partitionable:
        idx = np.arange(n, dtype=np.uint64)
        c1 = (idx >> np.uint64(32)).astype(_U32)
        c2 = (idx & np.uint64(0xFFFFFFFF)).astype(_U32)
        b1, b2 = _threefry2x32_np(k1, k2, c1, c2)
        return b1 ^ b2
    odd = n % 2
    cnt = np.arange(n + odd, dtype=_U32)
    half = len(cnt) // 2
    o1, o2 = _threefry2x32_np(k1, k2, cnt[:half], cnt[half:])
    out = np.concatenate([o1, o2])
    return out[:n] if odd else out


def _erfinv_np(x):
    x = np.asarray(x, np.float64)
    w = -np.log1p(-x * x)
    ws = w - 2.5
    p_small = np.float64(2.81022636e-08)
    for c in (3.43273939e-07, -3.5233877e-06, -4.39150654e-06, 0.00021858087,
              -0.00125372503, -0.00417768164, 0.246640727, 1.50140941):
        p_small = p_small * ws + c
    wl = np.sqrt(np.maximum(w, 1e-30)) - 3.0
    p_large = np.float64(-0.000200214257)
    for c in (0.000100950558, 0.00134934322, -0.00367342844, 0.00573950773,
              -0.0076224613, 0.00943887047, 1.00167406, 2.83297682):
        p_large = p_large * wl + c
    return np.where(w < 5.0, p_small, p_large) * x


def _shift_map_np(B, H, W):
    HP, WP = H // PATCH, W // PATCH
    n = B * HP * WP
    bits = _random_bits_np(0, 42, n)
    fb = (bits >> _U32(9)) | _U32(0x3F800000)
    f = fb.view(np.float32) - np.float32(1.0)
    lo = np.float32(np.nextafter(np.float32(-1.0), np.float32(0.0)))
    hi = np.float32(1.0)
    u = np.maximum(lo, (f * (hi - lo) + lo).astype(np.float32))
    normal = np.sqrt(2.0) * _erfinv_np(u.astype(np.float64))
    m = np.abs(normal * STDDEV)
    return m.astype(np.int32).reshape(B, HP, WP)


@functools.lru_cache(maxsize=None)
def _row_shifts(B, H, W, NW, NQ, UP):
    s_np = _shift_map_np(B, H, W)
    sv = np.repeat(s_np, PATCH, axis=1)
    RPT = H // NW
    out = np.zeros((NW, B, RPT, NQ, 16), np.int32)
    for t in range(NW):
        blk = sv[:, t * RPT:(t + 1) * RPT, :]
        out[t, :, :, :, :UP] = blk.reshape(B, RPT, NQ, UP)
    return out.reshape(-1)


def kernel(inputs):
    x = inputs
    B, H, W, C = x.shape
    info = plsc.get_sparse_core_info()
    NC, NS = info.num_cores, info.num_subcores
    NW = NC * NS
    RPT = H // NW
    PW = 128
    NQ = W // PW
    UP = PW // PATCH
    T = B * RPT * NQ
    shifts = _row_shifts(B, H, W, NW, NQ, UP)
    assert int(shifts.max()) < C

    xt = jnp.transpose(x, (0, 1, 3, 2))

    mesh = plsc.VectorSubcoreMesh(core_axis_name="c", subcore_axis_name="s")

    @functools.partial(
        pl.kernel,
        mesh=mesh,
        out_type=jax.ShapeDtypeStruct((B, H, C, W), jnp.float32),
        scratch_types=[
            pltpu.VMEM((B * RPT * NQ * 16,), jnp.int32),
            pltpu.VMEM((2, C, PW), jnp.float32),
            pltpu.VMEM((2, C, PW), jnp.float32),
            pltpu.SemaphoreType.DMA,
            pltpu.SemaphoreType.DMA,
        ],
    )
    def sc_rot(x_ref, vs_ref, out_ref, vsbuf, inb, outb, sem_in, sem_out):
        wid = lax.axis_index("s") * NC + lax.axis_index("c")
        h0 = wid * RPT
        NV = B * RPT * NQ * 16
        pltpu.sync_copy(vs_ref.at[pl.ds(wid * NV, NV)], vsbuf)

        def gather_piece(b, k, q, slot):
            pltpu.async_copy(
                x_ref.at[b, h0 + k, :, pl.ds(q * PW, PW)], inb.at[slot],
                sem_in)

        def wait_gather(slot):
            pltpu.make_async_copy(
                x_ref.at[0, 0, :, pl.ds(0, PW)], inb.at[slot], sem_in).wait()

        def wait_scatter(slot):
            pltpu.make_async_copy(
                outb.at[slot], out_ref.at[0, 0, :, pl.ds(0, PW)],
                sem_out).wait()

        def advance(b, k, q):
            qn = q + 1
            wrapq = qn >= NQ
            kn = jnp.where(wrapq, k + 1, k)
            qn = jnp.where(wrapq, 0, qn)
            wrapk = kn >= RPT
            bn = jnp.where(wrapk, b + 1, b)
            kn = jnp.where(wrapk, 0, kn)
            return bn, kn, qn

        gather_piece(0, 0, 0, 0)

        def piece(t, carry):
            b, k, q = carry
            nxt = advance(b, k, q)
            slot = lax.rem(t, 2)
            nslot = lax.rem(t + 1, 2)

            @pl.when(t >= 1)
            def _():
                wait_scatter(nslot)

            @pl.when(t + 1 < T)
            def _():
                gather_piece(nxt[0], nxt[1], nxt[2], nslot)

            wait_gather(slot)
            voff = ((b * RPT + k) * NQ + q) * 16
            vv = vsbuf[pl.ds(voff, 16)]
            svals = [vv[u] for u in range(UP)]

            @plsc.parallel_loop(0, C, unroll=8)
            def _(c):
                for u in range(UP):
                    src = c + svals[u]
                    src = jnp.where(src >= C, src - C, src)
                    outb[slot, c, pl.ds(u * PATCH, PATCH)] = (
                        inb[slot, src, pl.ds(u * PATCH, PATCH)])
            pltpu.async_copy(
                outb.at[slot],
                out_ref.at[b, h0 + k, :, pl.ds(q * PW, PW)], sem_out)
            return nxt

        lax.fori_loop(0, T, piece,
                      (jnp.int32(0), jnp.int32(0), jnp.int32(0)))
        wait_scatter(lax.rem(T - 1, 2))

    out = sc_rot(xt, jnp.asarray(shifts))
    return jnp.transpose(out, (0, 1, 3, 2))
